# Initial kernel scaffold; baseline (speedup 1.0000x reference)
import jax
import jax.numpy as jnp
from jax.experimental import pallas as pl


def kernel(node_feats, edge_feats, edge_index, graph_ids, W_proj, b_proj, W_e1, b_e1, W_e2, b_e2, b_conv, W_ih, W_hh, b_ih, b_hh, W_w, b_w, W_ff, b_ff):
    raise NotImplementedError("write your pallas kernel here")


# speedup vs baseline: 1.1257x; 1.1257x over previous
import functools

import jax
import jax.numpy as jnp
from jax import lax
from jax.experimental import pallas as pl
from jax.experimental.pallas import tpu as pltpu
from jax.experimental.pallas import tpu_sc as plsc

_N = 10000
_E = 20000
_H = 64
_EH = 32
_EHP = _EH + 1
_G = 256
_NCLS = 100
_STEPS = 3

_SC_CORES = 2
_SC_SUBCORES = 16
_NW = _SC_CORES * _SC_SUBCORES
_NP = 10240
_EP = 20480
_EPW = _EP // _NW
_ECH = _EPW // 128
_NPW = _NP // _SC_SUBCORES


def _sc_gather_body(h_hbm, src_hbm, out_hbm, idx_v, rows_v, sem):
    wid = lax.axis_index("s") * _SC_CORES + lax.axis_index("c")
    pltpu.sync_copy(src_hbm.at[pl.ds(wid * 8, 8)], idx_v)
    cps = [
        pltpu.async_copy(h_hbm.at[idx_v.at[j]],
                         rows_v.at[pl.ds(j * 128, 128)], sem)
        for j in range(_ECH)
    ]
    for cp in cps:
        cp.wait()
    pltpu.sync_copy(rows_v, out_hbm.at[pl.ds(wid * _EPW, _EPW)])


def _sc_scatter_body(m_hbm, dst_hbm, zero_hbm, out_hbm, idx_v, vals_v, acc_sh,
                     sem):
    cid = lax.axis_index("c")
    sid = lax.axis_index("s")
    wid = sid * _SC_CORES + cid
    ebase = wid * _EPW
    pltpu.sync_copy(zero_hbm.at[pl.ds(sid * _NPW, _NPW)],
                    acc_sh.at[pl.ds(sid * _NPW, _NPW)])
    pltpu.sync_copy(m_hbm.at[pl.ds(ebase, _EPW)], vals_v)
    pltpu.sync_copy(dst_hbm.at[pl.ds(wid * 8, 8)], idx_v)
    plsc.subcore_barrier()
    cps = [
        pltpu.async_copy(vals_v.at[pl.ds(j * 128, 128)],
                         acc_sh.at[idx_v.at[j]], sem, add=True)
        for j in range(_ECH)
    ]
    for cp in cps:
        cp.wait()
    plsc.subcore_barrier()
    pltpu.sync_copy(acc_sh.at[pl.ds(sid * _NPW, _NPW)],
                    out_hbm.at[pl.ds(cid * _NP + sid * _NPW, _NPW)])


@functools.lru_cache(maxsize=None)
def _sc_kernels():
    mesh = plsc.VectorSubcoreMesh(core_axis_name="c", subcore_axis_name="s")
    params = pltpu.CompilerParams(use_tc_tiling_on_sc=False)
    gather = pl.kernel(
        _sc_gather_body,
        out_type=jax.ShapeDtypeStruct((_EP, _H), jnp.float32),
        mesh=mesh,
        compiler_params=params,
        scratch_types=[
            pltpu.VMEM((8, 128), jnp.int32),
            pltpu.VMEM((_EPW, _H), jnp.float32),
            pltpu.SemaphoreType.DMA,
        ],
    )
    scatter = pl.kernel(
        _sc_scatter_body,
        out_type=jax.ShapeDtypeStruct((_SC_CORES * _NP, _H), jnp.float32),
        mesh=mesh,
        compiler_params=params,
        scratch_types=[
            pltpu.VMEM((8, 128), jnp.int32),
            pltpu.VMEM((_EPW, _H), jnp.float32),
            pltpu.VMEM_SHARED((_NP, _H), jnp.float32),
            pltpu.SemaphoreType.DMA,
        ],
    )
    return gather, scatter


def _dense_relu_body(x_ref, w_ref, b_ref, o_ref):
    o_ref[...] = jnp.maximum(x_ref[...] @ w_ref[...] + b_ref[...], 0.0)


def _re33_body(x_ref, w_ref, b_ref, o_ref):
    r = jnp.maximum(x_ref[...] @ w_ref[...] + b_ref[...], 0.0)
    ci = lax.broadcasted_iota(jnp.int32, r.shape, 1)
    o_ref[...] = jnp.where(ci < _EH, r, 1.0)


def _msg_body(hs_ref, re_ref, w2_ref, m_ref):
    p = hs_ref[...] @ w2_ref[...]
    re = re_ref[...]
    acc = re[:, 0:1] * p[:, 0:_H]
    for k in range(1, _EHP):
        acc = acc + re[:, k:k + 1] * p[:, k * _H:(k + 1) * _H]
    m_ref[...] = acc


def _gru_body(aa_ref, ab_ref, h_ref, bc_ref, wih_ref, whh_ref, bih_ref,
              bhh_ref, o_ref):
    x = jnp.maximum(aa_ref[...] + ab_ref[...] + bc_ref[...], 0.0)
    h = h_ref[...]
    gi = x @ wih_ref[...] + bih_ref[...]
    gh = h @ whh_ref[...] + bhh_ref[...]
    r = jax.nn.sigmoid(gi[:, 0:_H] + gh[:, 0:_H])
    z = jax.nn.sigmoid(gi[:, _H:2 * _H] + gh[:, _H:2 * _H])
    n = jnp.tanh(gi[:, 2 * _H:] + r * gh[:, 2 * _H:])
    o_ref[...] = (1.0 - z) * n + z * h


def _readout_body(h_ref, ids_ref, ww_ref, bw_ref, wff_ref, bff_ref, o_ref,
                  sacc, macc):
    i = pl.program_id(0)

    @pl.when(i == 0)
    def _():
        sacc[...] = jnp.zeros_like(sacc)
        macc[...] = jnp.full_like(macc, -jnp.inf)

    h = h_ref[...]
    ids = ids_ref[0]
    w = jax.nn.sigmoid(
        jnp.sum(h * ww_ref[...], axis=1, keepdims=True) + bw_ref[...])
    hw = h * w
    giota = lax.broadcasted_iota(jnp.int32, (_G, 512), 0)
    oh = (giota == ids).astype(jnp.float32)
    sacc[...] = sacc[...] + oh @ hw
    mcur = macc[...]
    for c in range(512 // _H):
        idc = ids[:, c * _H:(c + 1) * _H]
        ids3 = lax.broadcast_in_dim(idc, (_G, _H, _H), (0, 1))
        mask3 = lax.broadcasted_iota(jnp.int32, (_G, _H, _H), 0) == ids3
        hc = h[c * _H:(c + 1) * _H, :]
        h3 = lax.broadcast_in_dim(hc, (_G, _H, _H), (1, 2))
        masked = jnp.where(mask3, h3, -jnp.inf)
        mcur = jnp.maximum(mcur, jnp.max(masked, axis=1))
    macc[...] = mcur

    @pl.when(i == pl.num_programs(0) - 1)
    def _():
        hg = jnp.concatenate([sacc[...], macc[...]], axis=1)
        o_ref[...] = hg @ wff_ref[...] + bff_ref[...]


def _full(a):
    return pl.BlockSpec(a.shape, lambda i: tuple(0 for _ in a.shape))


def _dense_call(body, x, others, n_out, blk):
    n = x.shape[0]
    return pl.pallas_call(
        body,
        grid=(n // blk,),
        in_specs=[pl.BlockSpec((blk, x.shape[1]), lambda i: (i, 0))]
                 + [_full(a) for a in others],
        out_specs=pl.BlockSpec((blk, n_out), lambda i: (i, 0)),
        out_shape=jax.ShapeDtypeStruct((n, n_out), jnp.float32),
    )(x, *others)


def _msg_call(hs, re33, w2tp, blk):
    n = hs.shape[0]
    return pl.pallas_call(
        _msg_body,
        grid=(n // blk,),
        in_specs=[
            pl.BlockSpec((blk, _H), lambda i: (i, 0)),
            pl.BlockSpec((blk, _EHP), lambda i: (i, 0)),
            _full(w2tp),
        ],
        out_specs=pl.BlockSpec((blk, _H), lambda i: (i, 0)),
        out_shape=jax.ShapeDtypeStruct((n, _H), jnp.float32),
    )(hs, re33, w2tp)


def _gru_call(agg_a, agg_b, h, bc, wih, whh, bih, bhh, blk):
    n = h.shape[0]
    rowspec = pl.BlockSpec((blk, _H), lambda i: (i, 0))
    return pl.pallas_call(
        _gru_body,
        grid=(n // blk,),
        in_specs=[rowspec, rowspec, rowspec, _full(bc), _full(wih),
                  _full(whh), _full(bih), _full(bhh)],
        out_specs=rowspec,
        out_shape=jax.ShapeDtypeStruct((n, _H), jnp.float32),
    )(agg_a, agg_b, h, bc, wih, whh, bih, bhh)


def _readout_call(h, ids3, ww, bw, wff, bff):
    blk = 512
    return pl.pallas_call(
        _readout_body,
        grid=(_NP // blk,),
        in_specs=[
            pl.BlockSpec((blk, _H), lambda i: (i, 0)),
            pl.BlockSpec((1, 1, blk), lambda i: (i, 0, 0)),
            _full(ww), _full(bw), _full(wff), _full(bff),
        ],
        out_specs=pl.BlockSpec((_G, _NCLS), lambda i: (0, 0)),
        out_shape=jax.ShapeDtypeStruct((_G, _NCLS), jnp.float32),
        scratch_shapes=[
            pltpu.VMEM((_G, _H), jnp.float32),
            pltpu.VMEM((_G, _H), jnp.float32),
        ],
    )(h, ids3, ww, bw, wff, bff)


def kernel(node_feats, edge_feats, edge_index, graph_ids, W_proj, b_proj,
           W_e1, b_e1, W_e2, b_e2, b_conv, W_ih, W_hh, b_ih, b_hh, W_w, b_w,
           W_ff, b_ff):
    nf = jnp.pad(node_feats, ((0, _NP - _N), (0, 0)))
    ef = jnp.pad(edge_feats, ((0, _EP - _E), (0, 0)))
    src = jnp.pad(edge_index[0], (0, _EP - _E))
    dump = _N + (jnp.arange(_EP - _E, dtype=jnp.int32) % (_NP - _N))
    dst = jnp.concatenate([edge_index[1], dump])
    src2 = jnp.pad(src.reshape(_NW, _ECH, 128),
                   ((0, 0), (0, 8 - _ECH), (0, 0))).reshape(_NW * 8, 128)
    dst2 = jnp.pad(dst.reshape(_NW, _ECH, 128),
                   ((0, 0), (0, 8 - _ECH), (0, 0))).reshape(_NW * 8, 128)
    gids3 = jnp.pad(graph_ids, (0, _NP - _N),
                    constant_values=_G).reshape(_NP // 512, 1, 512)
    zeros_n = jnp.zeros((_NP, _H), jnp.float32)

    w_e1p = jnp.pad(W_e1, ((0, 0), (0, 1)))
    b_e1p = jnp.pad(b_e1, (0, 1)).reshape(1, _EHP)
    w2all = jnp.concatenate(
        [W_e2.reshape(_EH, _H, _H), b_e2.reshape(1, _H, _H)], axis=0)
    w2tp = w2all.transpose(1, 0, 2).reshape(_H, _EHP * _H)
    wih_t = W_ih.T
    whh_t = W_hh.T

    h0 = _dense_call(_dense_relu_body, nf,
                     [W_proj, b_proj.reshape(1, _H)], _H, 1024)
    re33 = _dense_call(_re33_body, ef, [w_e1p, b_e1p], _EHP, 1024)

    sc_gather, sc_scatter = _sc_kernels()
    h = h0
    hidden = h0
    for _ in range(_STEPS):
        hs = sc_gather(h, src2)
        m = _msg_call(hs, re33, w2tp, 1024)
        agg2 = sc_scatter(m, dst2, zeros_n)
        hidden = _gru_call(agg2[:_NP], agg2[_NP:], hidden,
                           b_conv.reshape(1, _H), wih_t, whh_t,
                           b_ih.reshape(1, 3 * _H), b_hh.reshape(1, 3 * _H),
                           1024)
        h = hidden

    return _readout_call(hidden, gids3, W_w.reshape(1, _H),
                         b_w.reshape(1, 1), W_ff, b_ff.reshape(1, _NCLS))
